# Initial kernel scaffold; baseline (speedup 1.0000x reference)
import jax
import jax.numpy as jnp
from jax.experimental import pallas as pl


def kernel(x, edge_index, batch, W1, att_src1, att_dst1, b1, W2, att_src2, att_dst2, b2, Wl, bl):
    raise NotImplementedError("write your pallas kernel here")


# speedup vs baseline: 1.0041x; 1.0041x over previous
import jax
import jax.numpy as jnp
from jax.experimental import pallas as pl

N_GRAPHS = 16


def _gat_conv(x, W, att_src, att_dst, bias, edge_index, heads, concat):
    N = x.shape[0]
    h = (x @ W).reshape(N, heads, -1)
    a_src = (h * att_src).sum(-1)
    a_dst = (h * att_dst).sum(-1)
    src = edge_index[0]
    dst = edge_index[1]
    alpha = jax.nn.leaky_relu(a_src[src] + a_dst[dst], negative_slope=0.2)
    amax = jax.ops.segment_max(alpha, dst, num_segments=N)
    amax = jnp.where(jnp.isfinite(amax), amax, 0.0)
    ex = jnp.exp(alpha - amax[dst])
    denom = jax.ops.segment_sum(ex, dst, num_segments=N)
    coef = ex / (denom[dst] + 1e-16)
    msg = h[src] * coef[:, :, None]
    out = jax.ops.segment_sum(msg, dst, num_segments=N)
    if concat:
        out = out.reshape(N, heads * out.shape[-1])
    else:
        out = out.mean(axis=1)
    return out + bias


def _pool_head_kernel(h_ref, batch_ref, Wl_ref, bl_ref, out_ref):
    h = h_ref[...]
    batch = batch_ref[...]
    gids = jax.lax.broadcasted_iota(jnp.int32, (N_GRAPHS, h.shape[0]), 0)
    onehot = (gids == batch[None, :]).astype(jnp.float32)
    sums = onehot @ h
    cnt = onehot.sum(axis=1, keepdims=True)
    pooled = sums / jnp.maximum(cnt, 1.0)
    out_ref[...] = pooled @ Wl_ref[...] + bl_ref[...][None, :]


def kernel(x, edge_index, batch, W1, att_src1, att_dst1, b1, W2, att_src2, att_dst2, b2, Wl, bl):
    h = _gat_conv(x, W1, att_src1, att_dst1, b1, edge_index, 4, concat=True)
    h = jax.nn.elu(h)
    h = _gat_conv(h, W2, att_src2, att_dst2, b2, edge_index, 1, concat=False)
    out = pl.pallas_call(
        _pool_head_kernel,
        out_shape=jax.ShapeDtypeStruct((N_GRAPHS, Wl.shape[1]), jnp.float32),
    )(h, batch, Wl, bl)
    return out


# speedup vs baseline: 1.7425x; 1.7354x over previous
import functools

import jax
import jax.numpy as jnp
from jax import lax
from jax.experimental import pallas as pl
from jax.experimental.pallas import tpu as pltpu
from jax.experimental.pallas import tpu_sc as plsc

N = 10000
NP = 10240
E = 160000
EP = 160256
D = 256
H1 = 4
C = 256
F1 = 1024
NG = 16
NC, NS, L = 2, 16, 16
NW = NC * NS
EW = EP // NW
EA = EP // NS
SENT = N
SL = NP // NS


def _mesh():
    return plsc.VectorSubcoreMesh(core_axis_name="c", subcore_axis_name="s")


_SC_PARAMS = pltpu.CompilerParams(needs_layout_passes=False)


def _mm_body(x_ref, w_ref, o_ref):
    o_ref[...] = jnp.dot(x_ref[...], w_ref[...], preferred_element_type=jnp.float32)


def _mm(x, w, rb):
    n, k = x.shape
    m = w.shape[1]
    return pl.pallas_call(
        _mm_body,
        grid=(n // rb,),
        in_specs=[pl.BlockSpec((rb, k), lambda i: (i, 0)),
                  pl.BlockSpec((k, m), lambda i: (0, 0))],
        out_specs=pl.BlockSpec((rb, m), lambda i: (i, 0)),
        out_shape=jax.ShapeDtypeStruct((n, m), jnp.float32),
    )(x, w)


def _mm_elu_body(x_ref, b_ref, w_ref, o_ref):
    a = x_ref[...] + b_ref[...]
    a = jnp.where(a > 0, a, jnp.exp(a) - 1.0)
    o_ref[...] = jnp.dot(a, w_ref[...], preferred_element_type=jnp.float32)


def _mm_elu(x, b, w, rb):
    n, k = x.shape
    m = w.shape[1]
    return pl.pallas_call(
        _mm_elu_body,
        grid=(n // rb,),
        in_specs=[pl.BlockSpec((rb, k), lambda i: (i, 0)),
                  pl.BlockSpec((1, k), lambda i: (0, 0)),
                  pl.BlockSpec((k, m), lambda i: (0, 0))],
        out_specs=pl.BlockSpec((rb, m), lambda i: (i, 0)),
        out_shape=jax.ShapeDtypeStruct((n, m), jnp.float32),
    )(x, b, w)


def _pool_body(h_ref, b2_ref, batch_ref, wl_ref, bl_ref, o_ref):
    h = h_ref[...] + b2_ref[...]
    batch = batch_ref[...]
    gids = lax.broadcasted_iota(jnp.int32, (NG, h.shape[0]), 0)
    onehot = (gids == batch[None, :]).astype(jnp.float32)
    sums = jnp.dot(onehot, h, preferred_element_type=jnp.float32)
    cnt = onehot.sum(axis=1, keepdims=True)
    pooled = sums / jnp.maximum(cnt, 1.0)
    o_ref[...] = jnp.dot(pooled, wl_ref[...], preferred_element_type=jnp.float32) + bl_ref[...][None, :]


def _pool_head(h, b2, batch, wl, bl):
    return pl.pallas_call(
        _pool_body,
        out_shape=jax.ShapeDtypeStruct((NG, wl.shape[1]), jnp.float32),
    )(h, b2, batch, wl, bl)


def _make_edge_stats(heads):
    @functools.partial(
        pl.kernel, mesh=_mesh(), compiler_params=_SC_PARAMS,
        out_type=[jax.ShapeDtypeStruct((heads * EP,), jnp.float32),
                  jax.ShapeDtypeStruct((NC * heads * NP,), jnp.float32)],
        scratch_types=[
            pltpu.VMEM((EW,), jnp.int32),
            pltpu.VMEM((EW,), jnp.int32),
            pltpu.VMEM((NP,), jnp.float32),
            pltpu.VMEM((NP,), jnp.float32),
            pltpu.VMEM((EW,), jnp.float32),
            pltpu.VMEM((NP,), jnp.float32),
            pltpu.VMEM((SL,), jnp.float32),
            pltpu.VMEM((SL,), jnp.float32),
            pltpu.VMEM_SHARED((NS * SL,), jnp.float32),
        ],
    )
    def k(srcp, dstp, asrc, adst, ex_o, dpart_o,
          src_v, dst_v, as_v, ad_v, ex_v, dacc, acc_v, tmp_v, shared):
        cid = lax.axis_index("c")
        sid = lax.axis_index("s")
        base = (cid * NS + sid) * EW
        pltpu.sync_copy(srcp.at[pl.ds(base, EW)], src_v)
        pltpu.sync_copy(dstp.at[pl.ds(base, EW)], dst_v)
        for h in range(heads):
            pltpu.sync_copy(asrc.at[pl.ds(h * NP, NP)], as_v)
            pltpu.sync_copy(adst.at[pl.ds(h * NP, NP)], ad_v)

            def zero_body(i, _):
                dacc[pl.ds(i * L, L)] = jnp.zeros((L,), jnp.float32)
                return 0
            lax.fori_loop(0, NP // L, zero_body, 0)

            def chunk_body(i, _):
                off = i * L
                is_ = src_v[pl.ds(off, L)]
                id_ = dst_v[pl.ds(off, L)]
                al = plsc.load_gather(as_v, [is_]) + plsc.load_gather(ad_v, [id_])
                al = jnp.where(al > 0, al, 0.2 * al)
                e = jnp.exp(al)
                ex_v[pl.ds(off, L)] = e
                plsc.addupdate_scatter(dacc, [id_], e)
                return 0
            lax.fori_loop(0, EW // L, chunk_body, 0)

            pltpu.sync_copy(ex_v, ex_o.at[pl.ds(h * EP + base, EW)])

            def round_body(r, _):
                pltpu.sync_copy(dacc.at[pl.ds(r * SL, SL)],
                                shared.at[pl.ds(sid * SL, SL)])
                plsc.subcore_barrier()

                @pl.when(sid == r)
                def _():
                    def zb(i, _):
                        acc_v[pl.ds(i * L, L)] = jnp.zeros((L,), jnp.float32)
                        return 0
                    lax.fori_loop(0, SL // L, zb, 0)

                    def tb(t, _):
                        pltpu.sync_copy(shared.at[pl.ds(t * SL, SL)], tmp_v)

                        def ab(i, _):
                            acc_v[pl.ds(i * L, L)] = (
                                acc_v[pl.ds(i * L, L)] + tmp_v[pl.ds(i * L, L)])
                            return 0
                        lax.fori_loop(0, SL // L, ab, 0)
                        return 0
                    lax.fori_loop(0, NS, tb, 0)
                    pltpu.sync_copy(
                        acc_v,
                        dpart_o.at[pl.ds((cid * heads + h) * NP + r * SL, SL)])
                plsc.subcore_barrier()
                return 0
            lax.fori_loop(0, NS, round_body, 0)
    return k


def _make_coef(heads):
    @functools.partial(
        pl.kernel, mesh=_mesh(), compiler_params=_SC_PARAMS,
        out_type=jax.ShapeDtypeStruct((heads * EP,), jnp.float32),
        scratch_types=[
            pltpu.VMEM((EW,), jnp.int32),
            pltpu.VMEM((EW,), jnp.float32),
            pltpu.VMEM((EW,), jnp.float32),
            pltpu.VMEM((NP,), jnp.float32),
            pltpu.VMEM((NP,), jnp.float32),
        ],
    )
    def k(dstp, ex_i, dpart, coef_o, dst_v, ex_v, cf_v, d0, d1):
        cid = lax.axis_index("c")
        sid = lax.axis_index("s")
        base = (cid * NS + sid) * EW
        pltpu.sync_copy(dstp.at[pl.ds(base, EW)], dst_v)
        for h in range(heads):
            pltpu.sync_copy(ex_i.at[pl.ds(h * EP + base, EW)], ex_v)
            pltpu.sync_copy(dpart.at[pl.ds(h * NP, NP)], d0)
            pltpu.sync_copy(dpart.at[pl.ds((heads + h) * NP, NP)], d1)

            def cb(i, _):
                off = i * L
                id_ = dst_v[pl.ds(off, L)]
                dn = plsc.load_gather(d0, [id_]) + plsc.load_gather(d1, [id_])
                cf_v[pl.ds(off, L)] = ex_v[pl.ds(off, L)] / (dn + 1e-16)
                return 0
            lax.fori_loop(0, EW // L, cb, 0)
            pltpu.sync_copy(cf_v, coef_o.at[pl.ds(h * EP + base, EW)])
    return k


def _make_agg_slab():
    SLW = 256
    CK = 512
    NCH = EP // CK
    RT = 5120 // NS

    @functools.partial(
        pl.kernel, mesh=_mesh(), compiler_params=_SC_PARAMS,
        out_type=jax.ShapeDtypeStruct((NP * 256,), jnp.float32),
        scratch_types=[
            pltpu.VMEM((RT * SLW,), jnp.float32),
            pltpu.VMEM((CK,), jnp.int32),
            pltpu.VMEM((CK,), jnp.int32),
            pltpu.VMEM((CK,), jnp.float32),
            pltpu.VMEM((CK + L,), jnp.int32),
            pltpu.VMEM((L, SLW), jnp.float32),
            pltpu.VMEM((L,), jnp.int32),
            pltpu.VMEM((L,), jnp.int32),
            pltpu.VMEM((L,), jnp.float32),
            pltpu.SemaphoreType.DMA,
        ],
    )
    def k(srcp, dstp, coefh, hrows, out_o,
          acc, sbuf, dbuf, cfbuf, eidx_v, rows_v, sidx_v, tvec_v, cbuf_v,
          sem):
        cid = lax.axis_index("c")
        sid = lax.axis_index("s")
        r0 = cid * 5120
        lanes = lax.broadcasted_iota(jnp.int32, (L,), 0)

        def zb(i, _):
            acc[pl.ds(i * L, L)] = jnp.zeros((L,), jnp.float32)
            return 0
        lax.fori_loop(0, RT * SLW // L, zb, 0)

        def chunk(c, _):
            pltpu.sync_copy(srcp.at[pl.ds(c * CK, CK)], sbuf)
            pltpu.sync_copy(dstp.at[pl.ds(c * CK, CK)], dbuf)
            pltpu.sync_copy(coefh.at[pl.ds(c * CK, CK)], cfbuf)

            def cb(i, nv):
                off = i * L
                d16 = dbuf[pl.ds(off, L)]
                loc = d16 - r0
                m = (loc >= 0) & (loc < 5120) & ((loc % NS) == sid)
                plsc.store_compressed(eidx_v.at[pl.ds(nv, L)], off + lanes,
                                      mask=m)
                return nv + jnp.sum(m.astype(jnp.int32))
            nv = lax.fori_loop(0, CK // L, cb, 0)
            ngrp = (nv + (L - 1)) // L

            def gb(j, _):
                el = eidx_v[pl.ds(j * L, L)]
                lm = (j * L + lanes) < nv
                el = jnp.where(lm, el, 0)
                s16 = plsc.load_gather(sbuf, [el])
                d16 = plsc.load_gather(dbuf, [el])
                c16 = plsc.load_gather(cfbuf, [el])
                sidx_v[...] = jnp.where(lm, s16, 0)
                tvec_v[...] = jnp.where(lm, (d16 - r0) // NS, 0)
                cbuf_v[...] = jnp.where(lm, c16, 0.0)
                pltpu.async_copy(hrows.at[sidx_v], rows_v, sem).wait()

                def rb(g, _):
                    gv = jnp.broadcast_to(g, (L,))
                    tgv = plsc.load_gather(tvec_v, [gv])
                    ch = plsc.load_gather(cbuf_v, [gv])

                    def sb(ii, _):
                        idx16 = tgv * SLW + ii * L + lanes
                        plsc.addupdate_scatter(
                            acc, [idx16], rows_v[g, pl.ds(ii * L, L)] * ch)
                        return 0
                    lax.fori_loop(0, SLW // L, sb, 0)
                    return 0
                lax.fori_loop(0, L, rb, 0)
                return 0
            lax.fori_loop(0, ngrp, gb, 0)
            return 0
        lax.fori_loop(0, NCH, chunk, 0)

        pltpu.sync_copy(
            acc, out_o.at[pl.ds((cid * 5120 + sid * RT) * SLW, RT * SLW)])
    return k


def _unpermute(perm):
    return perm.reshape(2, NS, 5120 // NS, 256).transpose(
        0, 2, 1, 3).reshape(NP, 256)


_edge_stats4 = _make_edge_stats(H1)
_edge_stats1 = _make_edge_stats(1)
_coef4 = _make_coef(H1)
_coef1 = _make_coef(1)
_aggslab = _make_agg_slab()


def _pad_att(a):
    return jnp.pad(a.T, ((0, 0), (0, NP - N))).reshape(-1)


@jax.jit
def kernel(x, edge_index, batch, W1, att_src1, att_dst1, b1,
           W2, att_src2, att_dst2, b2, Wl, bl):
    eyeh = jnp.eye(H1, dtype=jnp.float32)
    As1 = (att_src1[:, :, None] * eyeh[:, None, :]).reshape(F1, H1)
    Ad1 = (att_dst1[:, :, None] * eyeh[:, None, :]).reshape(F1, H1)
    W1e = jnp.concatenate([W1, W1 @ As1, W1 @ Ad1], axis=1)
    W1e = jnp.pad(W1e, ((0, 0), (0, 1152 - W1e.shape[1])))
    W2e = jnp.concatenate([W2, W2 @ att_src2.T, W2 @ att_dst2.T], axis=1)
    W2e = jnp.pad(W2e, ((0, 0), (0, 384 - W2e.shape[1])))

    src = edge_index[0]
    dst = edge_index[1]
    srcp = jnp.concatenate([src, jnp.zeros((EP - E,), jnp.int32)])
    dstp = jnp.concatenate([dst, jnp.full((EP - E,), SENT, jnp.int32)])

    o1 = _mm(x, W1e, 400)
    h1 = o1[:, :F1]
    a1s = _pad_att(o1[:, F1:F1 + H1])
    a1d = _pad_att(o1[:, F1 + H1:F1 + 2 * H1])

    ex1, dpart1 = _edge_stats4(srcp, dstp, a1s, a1d)
    coef1 = _coef4(dstp, ex1, dpart1)
    agg1 = jnp.concatenate(
        [_unpermute(_aggslab(srcp, dstp, coef1[h * EP:(h + 1) * EP],
                             o1[:, h * C:(h + 1) * C]))
         for h in range(H1)], axis=1)[:N]

    o2 = _mm_elu(agg1, b1.reshape(1, F1), W2e, 400)
    h2 = o2[:, :C]
    a2s = _pad_att(o2[:, C:C + 1])
    a2d = _pad_att(o2[:, C + 1:C + 2])

    ex2, dpart2 = _edge_stats1(srcp, dstp, a2s, a2d)
    coef2 = _coef1(dstp, ex2, dpart2)
    agg2 = _unpermute(_aggslab(srcp, dstp, coef2, h2))[:N]

    return _pool_head(agg2, b2.reshape(1, C), batch, Wl, bl)
